# baseline (speedup 1.0000x reference)
import functools

import jax
import jax.numpy as jnp
from jax import lax
from jax.experimental import pallas as pl
from jax.experimental.pallas import tpu as pltpu
from jax.experimental.pallas import tpu_sc as plsc

VOCAB = 1000000
EMBED = 32
INPUT_SIZE = 50
BATCH = 16384

NUM_CORES = 2
NUM_SUBCORES = 16
NW = NUM_CORES * NUM_SUBCORES
ROWS_PER_W = BATCH // NW
CHUNK = 32
NCHUNKS = ROWS_PER_W // CHUNK
IDX_PER_CHUNK = CHUNK * INPUT_SIZE
FLAT = INPUT_SIZE * EMBED
RB = 8
NB = CHUNK // RB


def _disc_body(x_hbm, tab_hbm, w_hbm, b_hbm, part_hbm,
               idx_a, idx_b, buf_a, buf_b, wv, bv, partials, sem_a, sem_b):
    cid = lax.axis_index("c")
    sid = lax.axis_index("s")
    wid = sid * NUM_CORES + cid

    pltpu.sync_copy(w_hbm, wv)
    pltpu.sync_copy(b_hbm, bv)

    def fire(idx_ref, buf_ref, sem, c):
        xoff = wid * (NCHUNKS * IDX_PER_CHUNK) + c * IDX_PER_CHUNK
        pltpu.sync_copy(x_hbm.at[pl.ds(xoff, IDX_PER_CHUNK)], idx_ref)
        pltpu.async_copy(tab_hbm.at[idx_ref], buf_ref, sem)

    def drain(buf_ref, sem):
        pltpu.make_async_copy(
            tab_hbm.at[pl.ds(0, IDX_PER_CHUNK)], buf_ref, sem
        ).wait()

    def compute(buf_ref, c):
        bias = bv[...]

        def blk_body(t, bcarry):
            r0 = t * RB

            def j_body(j, accs):
                w0 = wv[pl.ds(j * 32, 16)]
                w1 = wv[pl.ds(j * 32 + 16, 16)]
                out = []
                for rr in range(RB):
                    g = (r0 + rr) * INPUT_SIZE + j
                    out.append(accs[2 * rr] + buf_ref[g, pl.ds(0, 16)] * w0)
                    out.append(accs[2 * rr + 1] + buf_ref[g, pl.ds(16, 16)] * w1)
                return tuple(out)

            zero = jnp.zeros((16,), jnp.float32)
            accs = lax.fori_loop(0, INPUT_SIZE, j_body, (zero,) * (2 * RB))
            for rr in range(RB):
                partials[pl.ds((r0 + rr) * 16, 16)] = (
                    accs[2 * rr] + accs[2 * rr + 1] + bias
                )
            return bcarry

        lax.fori_loop(0, NB, blk_body, 0)
        row0 = wid * ROWS_PER_W + c * CHUNK
        pltpu.sync_copy(partials, part_hbm.at[pl.ds(row0 * 16, CHUNK * 16)])

    fire(idx_a, buf_a, sem_a, 0)

    def m_body(m, carry):
        fire(idx_b, buf_b, sem_b, 2 * m + 1)
        drain(buf_a, sem_a)
        compute(buf_a, 2 * m)

        @pl.when(m < NCHUNKS // 2 - 1)
        def _():
            fire(idx_a, buf_a, sem_a, 2 * m + 2)

        drain(buf_b, sem_b)
        compute(buf_b, 2 * m + 1)
        return carry

    lax.fori_loop(0, NCHUNKS // 2, m_body, 0)


def _finalize_body(p_ref, o_ref):
    z = jnp.sum(p_ref[...], axis=1, keepdims=True)
    o_ref[...] = 1.0 / (1.0 + jnp.exp(-z))


VCHUNK = 128
NFULL = VOCAB // VCHUNK
VTAIL = VOCAB - NFULL * VCHUNK
MPAIRS = (NFULL + 2 * NW - 1) // (2 * NW)
PSTRIDE = EMBED + 1
OUT_W = VCHUNK * EMBED


def _relayout_body(tt_hbm, tail_hbm, out_hbm,
                   ba0, ba1, ba2, ba3, bb0, bb1, bb2, bb3,
                   dpad, d3, tailbuf, sem_a, sem_b, sem_o):
    cid = lax.axis_index("c")
    sid = lax.axis_index("s")
    wid = sid * NUM_CORES + cid
    bufs_a = (ba0, ba1, ba2, ba3)
    bufs_b = (bb0, bb1, bb2, bb3)
    row16 = lax.iota(jnp.int32, 16)

    @pl.when(wid == 0)
    def _():
        pltpu.sync_copy(tail_hbm, tailbuf)
        pltpu.sync_copy(tailbuf, out_hbm.at[pl.ds(NFULL * OUT_W,
                                                  VTAIL * EMBED)])

    def cidx(j):
        return jnp.minimum(wid + NW * j, NFULL - 1)

    def fire(bufs, sem, j):
        c0 = pl.multiple_of(cidx(j) * VCHUNK, VCHUNK)
        for tr in range(4):
            pltpu.async_copy(
                tt_hbm.at[pl.ds(tr * 8, 8), pl.ds(c0, VCHUNK)], bufs[tr], sem
            )

    def drain_in(bufs, sem):
        for tr in range(4):
            pltpu.make_async_copy(
                tt_hbm.at[pl.ds(0, 8), pl.ds(0, VCHUNK)], bufs[tr], sem
            ).wait()

    def process(bufs, j, first):
        def g_body(g, gcarry):
            rows = g * 16 + row16
            for tr in range(4):
                for r in range(8):
                    k = tr * 8 + r
                    x16 = bufs[tr][r, pl.ds(g * 16, 16)]
                    plsc.store_scatter(
                        dpad, [rows, jnp.full((16,), k, jnp.int32)], x16
                    )
            return gcarry

        lax.fori_loop(0, VCHUNK // 16, g_body, 0)

        @pl.when(jnp.logical_not(first))
        def _():
            pltpu.make_async_copy(
                out_hbm.at[pl.ds(0, OUT_W)], d3, sem_o
            ).wait()

        def v_body(v4, vcarry):
            for vv in range(4):
                v = v4 * 4 + vv
                d3[pl.ds(v * EMBED, 16)] = dpad[v, pl.ds(0, 16)]
                d3[pl.ds(v * EMBED + 16, 16)] = dpad[v, pl.ds(16, 16)]
            return vcarry

        lax.fori_loop(0, VCHUNK // 4, v_body, 0)
        pltpu.async_copy(d3, out_hbm.at[pl.ds(cidx(j) * OUT_W, OUT_W)], sem_o)

    fire(bufs_a, sem_a, 0)

    def m_body(m, carry):
        fire(bufs_b, sem_b, 2 * m + 1)
        drain_in(bufs_a, sem_a)
        process(bufs_a, 2 * m, m == 0)

        @pl.when(m < MPAIRS - 1)
        def _():
            fire(bufs_a, sem_a, 2 * m + 2)

        drain_in(bufs_b, sem_b)
        process(bufs_b, 2 * m + 1, False)
        return carry

    lax.fori_loop(0, MPAIRS, m_body, 0)
    pltpu.make_async_copy(out_hbm.at[pl.ds(0, OUT_W)], d3, sem_o).wait()


def kernel(x, table, W, b):
    xf = x.astype(jnp.int32).reshape(BATCH * INPUT_SIZE)

    tail_lin = table[NFULL * VCHUNK:].reshape(VTAIL * EMBED)
    mesh = plsc.VectorSubcoreMesh(core_axis_name="c", subcore_axis_name="s")
    relayout = pl.kernel(
        _relayout_body,
        out_type=jax.ShapeDtypeStruct((VOCAB * EMBED,), jnp.float32),
        mesh=mesh,
        compiler_params=pltpu.CompilerParams(
            use_tc_tiling_on_sc=True, needs_layout_passes=False
        ),
        scratch_types=(
            [pltpu.VMEM((8, VCHUNK), jnp.float32)] * 8
            + [
                pltpu.VMEM((VCHUNK, PSTRIDE), jnp.float32),
                pltpu.VMEM((OUT_W,), jnp.float32),
                pltpu.VMEM((VTAIL * EMBED,), jnp.float32),
                pltpu.SemaphoreType.DMA,
                pltpu.SemaphoreType.DMA,
                pltpu.SemaphoreType.DMA,
            ]
        ),
    )
    t2 = relayout(table.T, tail_lin).reshape(VOCAB, EMBED)
    wf = W.reshape(FLAT).astype(jnp.float32)
    b16 = jnp.where(jnp.arange(16) == 0, b[0].astype(jnp.float32), 0.0)

    sc = pl.kernel(
        _disc_body,
        out_type=jax.ShapeDtypeStruct((BATCH * 16,), jnp.float32),
        mesh=mesh,
        compiler_params=pltpu.CompilerParams(
            use_tc_tiling_on_sc=False, needs_layout_passes=False
        ),
        scratch_types=[
            pltpu.VMEM((IDX_PER_CHUNK,), jnp.int32),
            pltpu.VMEM((IDX_PER_CHUNK,), jnp.int32),
            pltpu.VMEM((IDX_PER_CHUNK, EMBED), jnp.float32),
            pltpu.VMEM((IDX_PER_CHUNK, EMBED), jnp.float32),
            pltpu.VMEM((FLAT,), jnp.float32),
            pltpu.VMEM((16,), jnp.float32),
            pltpu.VMEM((CHUNK * 16,), jnp.float32),
            pltpu.SemaphoreType.DMA,
            pltpu.SemaphoreType.DMA,
        ],
    )
    partials = sc(xf, t2, wf, b16).reshape(BATCH, 16)

    blk = 2048
    out = pl.pallas_call(
        _finalize_body,
        out_shape=jax.ShapeDtypeStruct((BATCH, 1), jnp.float32),
        grid=(BATCH // blk,),
        in_specs=[pl.BlockSpec((blk, 16), lambda i: (i, 0))],
        out_specs=pl.BlockSpec((blk, 1), lambda i: (i, 0)),
    )(partials)
    return out


# speedup vs baseline: 1.0001x; 1.0001x over previous
import functools

import jax
import jax.numpy as jnp
from jax import lax
from jax.experimental import pallas as pl
from jax.experimental.pallas import tpu as pltpu
from jax.experimental.pallas import tpu_sc as plsc

VOCAB = 1000000
EMBED = 32
INPUT_SIZE = 50
BATCH = 16384

NUM_CORES = 2
NUM_SUBCORES = 16
NW = NUM_CORES * NUM_SUBCORES
ROWS_PER_W = BATCH // NW
CHUNK = 32
NCHUNKS = ROWS_PER_W // CHUNK
IDX_PER_CHUNK = CHUNK * INPUT_SIZE
FLAT = INPUT_SIZE * EMBED
RB = 8
NB = CHUNK // RB


def _disc_body(x_hbm, tab_hbm, w_hbm, b_hbm, part_hbm,
               idx_a, idx_b, buf_a, buf_b, wv, bv, partials, sem_a, sem_b):
    cid = lax.axis_index("c")
    sid = lax.axis_index("s")
    wid = sid * NUM_CORES + cid

    pltpu.sync_copy(w_hbm, wv)
    pltpu.sync_copy(b_hbm, bv)

    def fire(idx_ref, buf_ref, sem, c):
        xoff = wid * (NCHUNKS * IDX_PER_CHUNK) + c * IDX_PER_CHUNK
        pltpu.sync_copy(x_hbm.at[pl.ds(xoff, IDX_PER_CHUNK)], idx_ref)
        pltpu.async_copy(tab_hbm.at[idx_ref], buf_ref, sem)

    def drain(buf_ref, sem):
        pltpu.make_async_copy(
            tab_hbm.at[pl.ds(0, IDX_PER_CHUNK)], buf_ref, sem
        ).wait()

    def compute(buf_ref, c):
        bias = bv[...]

        def blk_body(t, bcarry):
            r0 = t * RB

            def j_body(j, accs):
                w0 = wv[pl.ds(j * 32, 16)]
                w1 = wv[pl.ds(j * 32 + 16, 16)]
                out = []
                for rr in range(RB):
                    g = (r0 + rr) * INPUT_SIZE + j
                    out.append(accs[2 * rr] + buf_ref[g, pl.ds(0, 16)] * w0)
                    out.append(accs[2 * rr + 1] + buf_ref[g, pl.ds(16, 16)] * w1)
                return tuple(out)

            zero = jnp.zeros((16,), jnp.float32)
            accs = lax.fori_loop(0, INPUT_SIZE, j_body, (zero,) * (2 * RB))
            for rr in range(RB):
                partials[pl.ds((r0 + rr) * 16, 16)] = (
                    accs[2 * rr] + accs[2 * rr + 1] + bias
                )
            return bcarry

        lax.fori_loop(0, NB, blk_body, 0)
        row0 = wid * ROWS_PER_W + c * CHUNK
        pltpu.sync_copy(partials, part_hbm.at[pl.ds(row0 * 16, CHUNK * 16)])

    fire(idx_a, buf_a, sem_a, 0)

    def m_body(m, carry):
        fire(idx_b, buf_b, sem_b, 2 * m + 1)
        drain(buf_a, sem_a)
        compute(buf_a, 2 * m)

        @pl.when(m < NCHUNKS // 2 - 1)
        def _():
            fire(idx_a, buf_a, sem_a, 2 * m + 2)

        drain(buf_b, sem_b)
        compute(buf_b, 2 * m + 1)
        return carry

    lax.fori_loop(0, NCHUNKS // 2, m_body, 0)


def _finalize_body(p_ref, o_ref):
    z = jnp.sum(p_ref[...], axis=1, keepdims=True)
    o_ref[...] = 1.0 / (1.0 + jnp.exp(-z))


VCHUNK = 128
NFULL = VOCAB // VCHUNK
VTAIL = VOCAB - NFULL * VCHUNK
MPAIRS = (NFULL + 2 * NW - 1) // (2 * NW)
PSTRIDE = EMBED + 1
OUT_W = VCHUNK * EMBED


def _relayout_body(tt_hbm, tail_hbm, out_hbm,
                   ba0, ba1, ba2, ba3, bb0, bb1, bb2, bb3,
                   dpad, d3a, d3b, tailbuf, sem_a, sem_b, sem_oa, sem_ob):
    cid = lax.axis_index("c")
    sid = lax.axis_index("s")
    wid = sid * NUM_CORES + cid
    bufs_a = (ba0, ba1, ba2, ba3)
    bufs_b = (bb0, bb1, bb2, bb3)
    row16 = lax.iota(jnp.int32, 16)

    @pl.when(wid == 0)
    def _():
        pltpu.sync_copy(tail_hbm, tailbuf)
        pltpu.sync_copy(tailbuf, out_hbm.at[pl.ds(NFULL * OUT_W,
                                                  VTAIL * EMBED)])

    def cidx(j):
        return jnp.minimum(wid + NW * j, NFULL - 1)

    def fire(bufs, sem, j):
        c0 = pl.multiple_of(cidx(j) * VCHUNK, VCHUNK)
        for tr in range(4):
            pltpu.async_copy(
                tt_hbm.at[pl.ds(tr * 8, 8), pl.ds(c0, VCHUNK)], bufs[tr], sem
            )

    def drain_in(bufs, sem):
        for tr in range(4):
            pltpu.make_async_copy(
                tt_hbm.at[pl.ds(0, 8), pl.ds(0, VCHUNK)], bufs[tr], sem
            ).wait()

    def process(bufs, d3, sem_o, j, first):
        def g_body(g, gcarry):
            rows = g * 16 + row16
            for tr in range(4):
                for r in range(8):
                    k = tr * 8 + r
                    x16 = bufs[tr][r, pl.ds(g * 16, 16)]
                    plsc.store_scatter(
                        dpad, [rows, jnp.full((16,), k, jnp.int32)], x16
                    )
            return gcarry

        lax.fori_loop(0, VCHUNK // 16, g_body, 0)

        @pl.when(jnp.logical_not(first))
        def _():
            pltpu.make_async_copy(
                out_hbm.at[pl.ds(0, OUT_W)], d3, sem_o
            ).wait()

        def v_body(v4, vcarry):
            for vv in range(4):
                v = v4 * 4 + vv
                d3[pl.ds(v * EMBED, 16)] = dpad[v, pl.ds(0, 16)]
                d3[pl.ds(v * EMBED + 16, 16)] = dpad[v, pl.ds(16, 16)]
            return vcarry

        lax.fori_loop(0, VCHUNK // 4, v_body, 0)
        pltpu.async_copy(d3, out_hbm.at[pl.ds(cidx(j) * OUT_W, OUT_W)], sem_o)

    fire(bufs_a, sem_a, 0)

    def m_body(m, carry):
        fire(bufs_b, sem_b, 2 * m + 1)
        drain_in(bufs_a, sem_a)
        process(bufs_a, d3a, sem_oa, 2 * m, m == 0)

        @pl.when(m < MPAIRS - 1)
        def _():
            fire(bufs_a, sem_a, 2 * m + 2)

        drain_in(bufs_b, sem_b)
        process(bufs_b, d3b, sem_ob, 2 * m + 1, m == 0)
        return carry

    lax.fori_loop(0, MPAIRS, m_body, 0)
    pltpu.make_async_copy(out_hbm.at[pl.ds(0, OUT_W)], d3a, sem_oa).wait()
    pltpu.make_async_copy(out_hbm.at[pl.ds(0, OUT_W)], d3b, sem_ob).wait()


def kernel(x, table, W, b):
    xf = x.astype(jnp.int32).reshape(BATCH * INPUT_SIZE)

    tail_lin = table[NFULL * VCHUNK:].reshape(VTAIL * EMBED)
    mesh = plsc.VectorSubcoreMesh(core_axis_name="c", subcore_axis_name="s")
    relayout = pl.kernel(
        _relayout_body,
        out_type=jax.ShapeDtypeStruct((VOCAB * EMBED,), jnp.float32),
        mesh=mesh,
        compiler_params=pltpu.CompilerParams(
            use_tc_tiling_on_sc=True, needs_layout_passes=False
        ),
        scratch_types=(
            [pltpu.VMEM((8, VCHUNK), jnp.float32)] * 8
            + [
                pltpu.VMEM((VCHUNK, PSTRIDE), jnp.float32),
                pltpu.VMEM((OUT_W,), jnp.float32),
                pltpu.VMEM((OUT_W,), jnp.float32),
                pltpu.VMEM((VTAIL * EMBED,), jnp.float32),
                pltpu.SemaphoreType.DMA,
                pltpu.SemaphoreType.DMA,
                pltpu.SemaphoreType.DMA,
                pltpu.SemaphoreType.DMA,
            ]
        ),
    )
    t2 = relayout(table.T, tail_lin).reshape(VOCAB, EMBED)
    wf = W.reshape(FLAT).astype(jnp.float32)
    b16 = jnp.where(jnp.arange(16) == 0, b[0].astype(jnp.float32), 0.0)

    sc = pl.kernel(
        _disc_body,
        out_type=jax.ShapeDtypeStruct((BATCH * 16,), jnp.float32),
        mesh=mesh,
        compiler_params=pltpu.CompilerParams(
            use_tc_tiling_on_sc=False, needs_layout_passes=False
        ),
        scratch_types=[
            pltpu.VMEM((IDX_PER_CHUNK,), jnp.int32),
            pltpu.VMEM((IDX_PER_CHUNK,), jnp.int32),
            pltpu.VMEM((IDX_PER_CHUNK, EMBED), jnp.float32),
            pltpu.VMEM((IDX_PER_CHUNK, EMBED), jnp.float32),
            pltpu.VMEM((FLAT,), jnp.float32),
            pltpu.VMEM((16,), jnp.float32),
            pltpu.VMEM((CHUNK * 16,), jnp.float32),
            pltpu.SemaphoreType.DMA,
            pltpu.SemaphoreType.DMA,
        ],
    )
    partials = sc(xf, t2, wf, b16).reshape(BATCH, 16)

    blk = 2048
    out = pl.pallas_call(
        _finalize_body,
        out_shape=jax.ShapeDtypeStruct((BATCH, 1), jnp.float32),
        grid=(BATCH // blk,),
        in_specs=[pl.BlockSpec((blk, 16), lambda i: (i, 0))],
        out_specs=pl.BlockSpec((blk, 1), lambda i: (i, 0)),
    )(partials)
    return out


# speedup vs baseline: 1.6065x; 1.6063x over previous
import functools

import jax
import jax.numpy as jnp
from jax import lax
from jax.experimental import pallas as pl
from jax.experimental.pallas import tpu as pltpu
from jax.experimental.pallas import tpu_sc as plsc

VOCAB = 1000000
EMBED = 32
INPUT_SIZE = 50
BATCH = 16384

NUM_CORES = 2
NUM_SUBCORES = 16
NW = NUM_CORES * NUM_SUBCORES
ROWS_PER_W = BATCH // NW
CHUNK = 32
NCHUNKS = ROWS_PER_W // CHUNK
IDX_PER_CHUNK = CHUNK * INPUT_SIZE
FLAT = INPUT_SIZE * EMBED
RB = 8
NB = CHUNK // RB


def _disc_body(x_hbm, tab_hbm, w_hbm, b_hbm, part_hbm,
               idx_a, idx_b, buf_a, buf_b, wv, bv, partials, sem_a, sem_b):
    cid = lax.axis_index("c")
    sid = lax.axis_index("s")
    wid = sid * NUM_CORES + cid

    pltpu.sync_copy(w_hbm, wv)
    pltpu.sync_copy(b_hbm, bv)

    def fire(idx_ref, buf_ref, sem, c):
        xoff = wid * (NCHUNKS * IDX_PER_CHUNK) + c * IDX_PER_CHUNK
        pltpu.sync_copy(x_hbm.at[pl.ds(xoff, IDX_PER_CHUNK)], idx_ref)
        pltpu.async_copy(tab_hbm.at[idx_ref], buf_ref, sem)

    def drain(buf_ref, sem):
        pltpu.make_async_copy(
            tab_hbm.at[pl.ds(0, IDX_PER_CHUNK)], buf_ref, sem
        ).wait()

    def compute(buf_ref, c):
        bias = bv[...]

        def blk_body(t, bcarry):
            r0 = t * RB

            def j_body(j, accs):
                w0 = wv[pl.ds(j * 32, 16)]
                w1 = wv[pl.ds(j * 32 + 16, 16)]
                out = []
                for rr in range(RB):
                    g = (r0 + rr) * INPUT_SIZE + j
                    out.append(accs[2 * rr] + buf_ref[g, pl.ds(0, 16)] * w0)
                    out.append(accs[2 * rr + 1] + buf_ref[g, pl.ds(16, 16)] * w1)
                return tuple(out)

            zero = jnp.zeros((16,), jnp.float32)
            accs = lax.fori_loop(0, INPUT_SIZE, j_body, (zero,) * (2 * RB))
            for rr in range(RB):
                partials[pl.ds((r0 + rr) * 16, 16)] = (
                    accs[2 * rr] + accs[2 * rr + 1] + bias
                )
            return bcarry

        lax.fori_loop(0, NB, blk_body, 0)
        row0 = wid * ROWS_PER_W + c * CHUNK
        pltpu.sync_copy(partials, part_hbm.at[pl.ds(row0 * 16, CHUNK * 16)])

    fire(idx_a, buf_a, sem_a, 0)

    def m_body(m, carry):
        fire(idx_b, buf_b, sem_b, 2 * m + 1)
        drain(buf_a, sem_a)
        compute(buf_a, 2 * m)

        @pl.when(m < NCHUNKS // 2 - 1)
        def _():
            fire(idx_a, buf_a, sem_a, 2 * m + 2)

        drain(buf_b, sem_b)
        compute(buf_b, 2 * m + 1)
        return carry

    lax.fori_loop(0, NCHUNKS // 2, m_body, 0)


def _finalize_body(p_ref, o_ref):
    z = jnp.sum(p_ref[...], axis=1, keepdims=True)
    o_ref[...] = 1.0 / (1.0 + jnp.exp(-z))


def kernel(x, table, W, b):
    xf = x.astype(jnp.int32).reshape(BATCH * INPUT_SIZE)
    t128 = lax.optimization_barrier(table.reshape(VOCAB // 4, EMBED * 4))
    t2 = t128.reshape(VOCAB, EMBED)
    wf = W.reshape(FLAT).astype(jnp.float32)
    b16 = jnp.where(jnp.arange(16) == 0, b[0].astype(jnp.float32), 0.0)

    mesh = plsc.VectorSubcoreMesh(core_axis_name="c", subcore_axis_name="s")
    sc = pl.kernel(
        _disc_body,
        out_type=jax.ShapeDtypeStruct((BATCH * 16,), jnp.float32),
        mesh=mesh,
        compiler_params=pltpu.CompilerParams(
            use_tc_tiling_on_sc=False, needs_layout_passes=False
        ),
        scratch_types=[
            pltpu.VMEM((IDX_PER_CHUNK,), jnp.int32),
            pltpu.VMEM((IDX_PER_CHUNK,), jnp.int32),
            pltpu.VMEM((IDX_PER_CHUNK, EMBED), jnp.float32),
            pltpu.VMEM((IDX_PER_CHUNK, EMBED), jnp.float32),
            pltpu.VMEM((FLAT,), jnp.float32),
            pltpu.VMEM((16,), jnp.float32),
            pltpu.VMEM((CHUNK * 16,), jnp.float32),
            pltpu.SemaphoreType.DMA,
            pltpu.SemaphoreType.DMA,
        ],
    )
    partials = sc(xf, t2, wf, b16).reshape(BATCH, 16)

    blk = 2048
    out = pl.pallas_call(
        _finalize_body,
        out_shape=jax.ShapeDtypeStruct((BATCH, 1), jnp.float32),
        grid=(BATCH // blk,),
        in_specs=[pl.BlockSpec((blk, 16), lambda i: (i, 0))],
        out_specs=pl.BlockSpec((blk, 1), lambda i: (i, 0)),
    )(partials)
    return out
